# baseline (speedup 1.0000x reference)
import jax
import jax.numpy as jnp
from jax.experimental import pallas as pl
from jax.experimental.pallas import tpu as pltpu

_G = 8
_EG = 8
_E = _G * _EG
_K = 2
_BLOCK = 2048


def _route(wc, x):
    lt = jax.lax.dot_general(wc, x, (((1,), (1,)), ((), ())),
                             preferred_element_type=jnp.float32)
    lg = lt[0:_G, :]
    le = lt[_G:_G + _E, :]
    b = x.shape[0]

    gm = jnp.max(lg, axis=0, keepdims=True)
    gpu = jnp.exp(lg - gm)
    gp = gpu / jnp.sum(gpu, axis=0, keepdims=True)

    m = jnp.max(le, axis=0, keepdims=True)
    p0 = jnp.exp(le - m)
    s_parts = []
    g_parts = []
    for g in range(_G):
        blk = p0[g * _EG:(g + 1) * _EG, :]
        sg = jnp.sum(blk, axis=0, keepdims=True)
        s_parts.append(jnp.broadcast_to(sg, (_EG, b)))
        g_parts.append(jnp.broadcast_to(gp[g:g + 1, :], (_EG, b)))
    s = jnp.concatenate(s_parts, axis=0)
    gpb = jnp.concatenate(g_parts, axis=0)
    ep = p0 / s

    w = gpb * ep
    vmask = jnp.where((gpb >= (1.0 / _G)) & (ep >= (1.0 / _EG)), 1.0, 0.0)
    nsel = jnp.sum(vmask, axis=0, keepdims=True)

    sub = jax.lax.broadcasted_iota(jnp.int32, w.shape, 0)
    m1 = jnp.max(w, axis=0, keepdims=True)
    i1 = jnp.min(jnp.where(w == m1, sub, _E), axis=0, keepdims=True)
    w2 = jnp.where(sub == i1, -1.0, w)
    m2 = jnp.max(w2, axis=0, keepdims=True)
    i2 = jnp.min(jnp.where(w2 == m2, sub, _E), axis=0, keepdims=True)
    tmask = jnp.where((sub == i1) | (sub == i2), 1.0, 0.0)

    fmask = jnp.where(nsel < float(_K), tmask, vmask)
    sw = w * fmask
    ws = jnp.maximum(jnp.sum(sw, axis=0, keepdims=True), 1e-9)
    return (sw / ws).T, fmask.T


def _router_kernel(wc_ref, xlo_ref, xhi_ref, mask_ref, w_ref):
    wc = wc_ref[...]
    nw_lo, fm_lo = _route(wc, xlo_ref[0])
    nw_hi, fm_hi = _route(wc, xhi_ref[0])
    w_ref[0] = nw_lo
    w_ref[1] = nw_hi
    mask_ref[0] = fm_lo.astype(jnp.int8)
    mask_ref[1] = fm_hi.astype(jnp.int8)


@jax.jit
def kernel(x, Wg, We):
    n, d = x.shape
    half = n // 2
    wc = jnp.concatenate([Wg, We], axis=0)
    x3 = x.reshape(2, half, d)
    mask, w = pl.pallas_call(
        _router_kernel,
        grid=(half // _BLOCK,),
        in_specs=[
            pl.BlockSpec((_G + _E, d), lambda i: (0, 0)),
            pl.BlockSpec((1, _BLOCK, d), lambda i: (0, i, 0)),
            pl.BlockSpec((1, _BLOCK, d), lambda i: (1, i, 0)),
        ],
        out_specs=[
            pl.BlockSpec((2, _BLOCK, _E), lambda i: (0, i, 0)),
            pl.BlockSpec((2, _BLOCK, _E), lambda i: (0, i, 0)),
        ],
        out_shape=[
            jax.ShapeDtypeStruct((2, half, _E), jnp.int8),
            jax.ShapeDtypeStruct((2, half, _E), jnp.float32),
        ],
        compiler_params=pltpu.CompilerParams(
            dimension_semantics=("parallel",)),
    )(wc, x3, x3)
    return (mask.reshape(n, _E).astype(jnp.bool_), w.reshape(n, _E))


# speedup vs baseline: 1.0145x; 1.0145x over previous
import jax
import jax.numpy as jnp
from jax.experimental import pallas as pl
from jax.experimental.pallas import tpu as pltpu

_G = 8
_EG = 8
_E = _G * _EG
_K = 2
_BLOCK = 4096


def _router_kernel(wc_ref, x_ref, mask_ref, w_ref):
    x = x_ref[...]
    lt = jax.lax.dot_general(wc_ref[...], x, (((1,), (1,)), ((), ())),
                             preferred_element_type=jnp.float32)
    lg = lt[0:_G, :]
    le = lt[_G:_G + _E, :]
    b = x.shape[0]

    gm = jnp.max(lg, axis=0, keepdims=True)
    gpu = jnp.exp(lg - gm)
    gp = gpu / jnp.sum(gpu, axis=0, keepdims=True)

    m = jnp.max(le, axis=0, keepdims=True)
    p0 = jnp.exp(le - m)
    s_parts = []
    g_parts = []
    for g in range(_G):
        blk = p0[g * _EG:(g + 1) * _EG, :]
        sg = jnp.sum(blk, axis=0, keepdims=True)
        s_parts.append(jnp.broadcast_to(sg, (_EG, b)))
        g_parts.append(jnp.broadcast_to(gp[g:g + 1, :], (_EG, b)))
    s = jnp.concatenate(s_parts, axis=0)
    gpb = jnp.concatenate(g_parts, axis=0)
    ep = p0 / s

    w = gpb * ep
    vmask = jnp.where((gpb >= (1.0 / _G)) & (ep >= (1.0 / _EG)), 1.0, 0.0)
    nsel = jnp.sum(vmask, axis=0, keepdims=True)

    sub = jax.lax.broadcasted_iota(jnp.int32, w.shape, 0)
    m1 = jnp.max(w, axis=0, keepdims=True)
    i1 = jnp.min(jnp.where(w == m1, sub, _E), axis=0, keepdims=True)
    w2 = jnp.where(sub == i1, -1.0, w)
    m2 = jnp.max(w2, axis=0, keepdims=True)
    i2 = jnp.min(jnp.where(w2 == m2, sub, _E), axis=0, keepdims=True)
    tmask = jnp.where((sub == i1) | (sub == i2), 1.0, 0.0)

    fmask = jnp.where(nsel < float(_K), tmask, vmask)
    sw = w * fmask
    ws = jnp.maximum(jnp.sum(sw, axis=0, keepdims=True), 1e-9)
    w_ref[...] = (sw / ws).T
    mask_ref[...] = fmask.T > 0.5


@jax.jit
def kernel(x, Wg, We):
    n, d = x.shape
    wc = jnp.concatenate([Wg, We], axis=0)
    mask, w = pl.pallas_call(
        _router_kernel,
        grid=(n // _BLOCK,),
        in_specs=[
            pl.BlockSpec((_G + _E, d), lambda i: (0, 0)),
            pl.BlockSpec((_BLOCK, d), lambda i: (i, 0)),
        ],
        out_specs=[
            pl.BlockSpec((_BLOCK, _E), lambda i: (i, 0)),
            pl.BlockSpec((_BLOCK, _E), lambda i: (i, 0)),
        ],
        out_shape=[
            jax.ShapeDtypeStruct((n, _E), jnp.bool_),
            jax.ShapeDtypeStruct((n, _E), jnp.float32),
        ],
        compiler_params=pltpu.CompilerParams(
            dimension_semantics=("parallel",)),
    )(wc, x)
    return mask, w


# speedup vs baseline: 1.0946x; 1.0789x over previous
import jax
import jax.numpy as jnp
from jax.experimental import pallas as pl
from jax.experimental.pallas import tpu as pltpu

_G = 8
_EG = 8
_E = _G * _EG
_K = 2
_BLOCK = 4096


def _router_kernel(wc_ref, x_ref, mask_ref, w_ref):
    x = x_ref[...]
    lt = jax.lax.dot_general(wc_ref[...], x, (((1,), (1,)), ((), ())),
                             preferred_element_type=jnp.float32)
    lg = lt[0:_G, :]
    le = lt[_G:_G + _E, :]
    b = x.shape[0]

    gm = jnp.max(lg, axis=0, keepdims=True)
    gpu = jnp.exp(lg - gm)
    gp = gpu / jnp.sum(gpu, axis=0, keepdims=True)

    m = jnp.max(le, axis=0, keepdims=True)
    p0 = jnp.exp(le - m)
    s_parts = []
    g_parts = []
    for g in range(_G):
        blk = p0[g * _EG:(g + 1) * _EG, :]
        sg = jnp.sum(blk, axis=0, keepdims=True)
        s_parts.append(jnp.broadcast_to(sg, (_EG, b)))
        g_parts.append(jnp.broadcast_to(gp[g:g + 1, :], (_EG, b)))
    s = jnp.concatenate(s_parts, axis=0)
    gpb = jnp.concatenate(g_parts, axis=0)
    ep = p0 / s

    w = gpb * ep
    vmask = jnp.where((gpb >= (1.0 / _G)) & (ep >= (1.0 / _EG)), 1.0, 0.0)
    nsel = jnp.sum(vmask, axis=0, keepdims=True)

    sub = jax.lax.broadcasted_iota(jnp.int32, w.shape, 0)
    m1 = jnp.max(w, axis=0, keepdims=True)
    i1 = jnp.min(jnp.where(w == m1, sub, _E), axis=0, keepdims=True)
    w2 = jnp.where(sub == i1, -1.0, w)
    m2 = jnp.max(w2, axis=0, keepdims=True)
    i2 = jnp.min(jnp.where(w2 == m2, sub, _E), axis=0, keepdims=True)
    tmask = jnp.where((sub == i1) | (sub == i2), 1.0, 0.0)

    fmask = jnp.where(nsel < float(_K), tmask, vmask)
    sw = w * fmask
    ws = jnp.maximum(jnp.sum(sw, axis=0, keepdims=True), 1e-9)
    w_ref[...] = (sw / ws).T
    mask_ref[...] = fmask.T.astype(jnp.int8)


@jax.jit
def kernel(x, Wg, We):
    n, d = x.shape
    wc = jnp.concatenate([Wg, We], axis=0)
    mask, w = pl.pallas_call(
        _router_kernel,
        grid=(n // _BLOCK,),
        in_specs=[
            pl.BlockSpec((_G + _E, d), lambda i: (0, 0)),
            pl.BlockSpec((_BLOCK, d), lambda i: (i, 0)),
        ],
        out_specs=[
            pl.BlockSpec((_BLOCK, _E), lambda i: (i, 0)),
            pl.BlockSpec((_BLOCK, _E), lambda i: (i, 0)),
        ],
        out_shape=[
            jax.ShapeDtypeStruct((n, _E), jnp.int8),
            jax.ShapeDtypeStruct((n, _E), jnp.float32),
        ],
        compiler_params=pltpu.CompilerParams(
            dimension_semantics=("parallel",)),
    )(wc, x)
    return mask.astype(jnp.bool_), w
